# Initial kernel scaffold; baseline (speedup 1.0000x reference)
import jax
import jax.numpy as jnp
from jax.experimental import pallas as pl


def kernel(z_e, embeddings):
    raise NotImplementedError("write your pallas kernel here")


# speedup vs baseline: 1.5171x; 1.5171x over previous
import functools

import jax
import jax.numpy as jnp
from jax import lax
from jax.experimental import pallas as pl
from jax.experimental.pallas import tpu as pltpu
from jax.experimental.pallas import tpu_sc as plsc

B = 16384
K = 8192
D = 64
BT = 256
NB = B // BT
EPS = 1e-12

NC = 2
NS = 16
NW = NC * NS
BPW = B // NW
CH = 128
NCH = BPW // CH


def _main_body(z_ref, emb_ref, embt_ref, zn_ref, en_ref, idx_ref):
    i = pl.program_id(0)
    z = z_ref[...]
    zn = z / jnp.maximum(jnp.sqrt(jnp.sum(z * z, axis=1, keepdims=True)), EPS)
    zn_ref[...] = zn

    @pl.when(i == 0)
    def _():
        e = emb_ref[...]
        en_ref[...] = e / jnp.maximum(
            jnp.sqrt(jnp.sum(e * e, axis=1, keepdims=True)), EPS)

    et = embt_ref[...]
    etn = et / jnp.maximum(jnp.sqrt(jnp.sum(et * et, axis=0, keepdims=True)), EPS)
    sim = jnp.dot(zn, etn, preferred_element_type=jnp.float32)
    m = jnp.max(sim, axis=1, keepdims=True)
    ii = lax.broadcasted_iota(jnp.int32, sim.shape, 1)
    idx = jnp.min(jnp.where(sim == m, ii, K), axis=1)
    idx_ref[...] = idx.reshape(1, 1, BT)


def _make_main(interpret=False):
    return pl.pallas_call(
        _main_body,
        grid=(NB,),
        in_specs=[
            pl.BlockSpec((BT, D), lambda i: (i, 0)),
            pl.BlockSpec((K, D), lambda i: (0, 0)),
            pl.BlockSpec((D, K), lambda i: (0, 0)),
        ],
        out_specs=[
            pl.BlockSpec((BT, D), lambda i: (i, 0)),
            pl.BlockSpec((K, D), lambda i: (0, 0)),
            pl.BlockSpec((1, 1, BT), lambda i: (i, 0, 0)),
        ],
        out_shape=[
            jax.ShapeDtypeStruct((B, D), jnp.float32),
            jax.ShapeDtypeStruct((K, D), jnp.float32),
            jax.ShapeDtypeStruct((NB, 1, BT), jnp.int32),
        ],
        interpret=interpret,
    )


def _sc_body(tab_ref, idx_ref, zq_ref, cnt_ref, idxv, rows, ones_v, zer_v,
             cnt_sh, sem):
    cid = lax.axis_index("c")
    sid = lax.axis_index("s")
    wid = sid * NC + cid
    pltpu.sync_copy(idx_ref.at[pl.ds(wid * NCH, NCH)], idxv)
    for j in range(NCH):
        pltpu.async_copy(tab_ref.at[idxv.at[j]],
                         rows.at[pl.ds(j * CH, CH)], sem).wait()
    pltpu.sync_copy(rows, zq_ref.at[pl.ds(wid * BPW, BPW)])
    for j in range(CH // 16):
        ones_v[pl.ds(j * 16, 16)] = jnp.ones((16,), jnp.float32)
    for j in range((K // NS) // 16):
        zer_v[pl.ds(j * 16, 16)] = jnp.zeros((16,), jnp.float32)
    pltpu.sync_copy(zer_v, cnt_sh.at[pl.ds(sid * (K // NS), K // NS)])
    plsc.subcore_barrier()
    for j in range(NCH):
        pltpu.sync_copy(ones_v, cnt_sh.at[idxv.at[j]], add=True)
    plsc.subcore_barrier()

    @pl.when(sid == 0)
    def _():
        pltpu.sync_copy(cnt_sh, cnt_ref.at[cid])


def _make_sc():
    mesh = plsc.VectorSubcoreMesh(core_axis_name="c", subcore_axis_name="s")
    return pl.kernel(
        _sc_body,
        mesh=mesh,
        out_type=[
            jax.ShapeDtypeStruct((B, D), jnp.float32),
            jax.ShapeDtypeStruct((NC, K), jnp.float32),
        ],
        scratch_types=[
            pltpu.VMEM((NCH, CH), jnp.int32),
            pltpu.VMEM((BPW, D), jnp.float32),
            pltpu.VMEM((CH,), jnp.float32),
            pltpu.VMEM((K // NS,), jnp.float32),
            pltpu.VMEM_SHARED((K,), jnp.float32),
            pltpu.SemaphoreType.DMA,
        ],
        compiler_params=pltpu.CompilerParams(use_tc_tiling_on_sc=False),
    )


def _fin_body(zn_ref, zq_ref, cnt_ref, zste_ref, com_ref, cod_ref, per_ref,
              util_ref):
    zn = zn_ref[...]
    zq = zq_ref[...]
    zste_ref[...] = zn + (zq - zn)
    dlt = zn - zq
    mse = jnp.sum(dlt * dlt) * (1.0 / (B * D))
    com_ref[0, 0] = 0.25 * mse
    cod_ref[0, 0] = mse
    c = cnt_ref[...]
    cc = c[0:1, :] + c[1:2, :]
    p = cc * (1.0 / B)
    ent = -jnp.sum(p * jnp.log(p + 1e-10))
    per_ref[0, 0] = jnp.exp(ent)
    util_ref[0, 0] = jnp.sum((cc > 0.0).astype(jnp.float32)) * (1.0 / K)


def _make_fin(interpret=False):
    scalar = jax.ShapeDtypeStruct((1, 1), jnp.float32)
    return pl.pallas_call(
        _fin_body,
        grid=(1,),
        in_specs=[
            pl.BlockSpec((B, D), lambda i: (0, 0)),
            pl.BlockSpec((B, D), lambda i: (0, 0)),
            pl.BlockSpec((NC, K), lambda i: (0, 0)),
        ],
        out_specs=[
            pl.BlockSpec((B, D), lambda i: (0, 0)),
            pl.BlockSpec((1, 1), lambda i: (0, 0), memory_space=pltpu.SMEM),
            pl.BlockSpec((1, 1), lambda i: (0, 0), memory_space=pltpu.SMEM),
            pl.BlockSpec((1, 1), lambda i: (0, 0), memory_space=pltpu.SMEM),
            pl.BlockSpec((1, 1), lambda i: (0, 0), memory_space=pltpu.SMEM),
        ],
        out_shape=[
            jax.ShapeDtypeStruct((B, D), jnp.float32),
            scalar, scalar, scalar, scalar,
        ],
        interpret=interpret,
    )


_main = _make_main()
_fin = _make_fin()
_get_sc = functools.cache(_make_sc)


def kernel(z_e, embeddings):
    zn, en, idx3 = _main(z_e, embeddings, embeddings.T)
    idx2 = idx3.reshape(B // CH, CH)
    zq, cnt2 = _get_sc()(en, idx2)
    zste, com, cod, per, util = _fin(zn, zq, cnt2)
    return (zste, idx3.reshape(B), com.reshape(()), cod.reshape(()),
            per.reshape(()), util.reshape(()))


# speedup vs baseline: 2.1472x; 1.4154x over previous
import functools

import jax
import jax.numpy as jnp
from jax import lax
from jax.experimental import pallas as pl
from jax.experimental.pallas import tpu as pltpu
from jax.experimental.pallas import tpu_sc as plsc

B = 16384
K = 8192
D = 64
BT = 256
NB = B // BT
EPS = 1e-12

NC = 2
NS = 16
NW = NC * NS
BPW = B // NW
CH = 128
NCH = BPW // CH


def _prep_body(emb_ref, embt_ref, en_ref, etn_ref):
    e = emb_ref[...]
    en_ref[...] = e / jnp.maximum(
        jnp.sqrt(jnp.sum(e * e, axis=1, keepdims=True)), EPS)
    et = embt_ref[...]
    etn_ref[...] = et / jnp.maximum(
        jnp.sqrt(jnp.sum(et * et, axis=0, keepdims=True)), EPS)


def _make_prep(interpret=False):
    return pl.pallas_call(
        _prep_body,
        out_shape=[
            jax.ShapeDtypeStruct((K, D), jnp.float32),
            jax.ShapeDtypeStruct((D, K), jnp.float32),
        ],
        interpret=interpret,
    )


def _main_body(z_ref, etn_ref, zn_ref, idx_ref):
    z = z_ref[...]
    zn = z / jnp.maximum(jnp.sqrt(jnp.sum(z * z, axis=1, keepdims=True)), EPS)
    zn_ref[...] = zn
    etn = etn_ref[...]
    sim = jnp.dot(zn, etn, preferred_element_type=jnp.float32)
    idx = jnp.argmax(sim, axis=1).astype(jnp.int32)
    idx_ref[...] = idx.reshape(1, 1, BT)


def _make_main(interpret=False):
    return pl.pallas_call(
        _main_body,
        grid=(NB,),
        in_specs=[
            pl.BlockSpec((BT, D), lambda i: (i, 0)),
            pl.BlockSpec((D, K), lambda i: (0, 0)),
        ],
        out_specs=[
            pl.BlockSpec((BT, D), lambda i: (i, 0)),
            pl.BlockSpec((1, 1, BT), lambda i: (i, 0, 0)),
        ],
        out_shape=[
            jax.ShapeDtypeStruct((B, D), jnp.float32),
            jax.ShapeDtypeStruct((NB, 1, BT), jnp.int32),
        ],
        interpret=interpret,
    )


def _sc_body(tab_ref, idx_ref, zq_ref, cnt_ref, idxv, rows, ones_v, zer_v,
             cnt_sh, sem):
    cid = lax.axis_index("c")
    sid = lax.axis_index("s")
    wid = sid * NC + cid
    pltpu.sync_copy(idx_ref.at[pl.ds(wid * NCH, NCH)], idxv)
    for j in range(NCH):
        pltpu.async_copy(tab_ref.at[idxv.at[j]],
                         rows.at[pl.ds(j * CH, CH)], sem).wait()
    pltpu.sync_copy(rows, zq_ref.at[pl.ds(wid * BPW, BPW)])
    for j in range(CH // 16):
        ones_v[pl.ds(j * 16, 16)] = jnp.ones((16,), jnp.float32)
    for j in range((K // NS) // 16):
        zer_v[pl.ds(j * 16, 16)] = jnp.zeros((16,), jnp.float32)
    pltpu.sync_copy(zer_v, cnt_sh.at[pl.ds(sid * (K // NS), K // NS)])
    plsc.subcore_barrier()
    for j in range(NCH):
        pltpu.sync_copy(ones_v, cnt_sh.at[idxv.at[j]], add=True)
    plsc.subcore_barrier()

    @pl.when(sid == 0)
    def _():
        pltpu.sync_copy(cnt_sh, cnt_ref.at[cid])


def _make_sc():
    mesh = plsc.VectorSubcoreMesh(core_axis_name="c", subcore_axis_name="s")
    return pl.kernel(
        _sc_body,
        mesh=mesh,
        out_type=[
            jax.ShapeDtypeStruct((B, D), jnp.float32),
            jax.ShapeDtypeStruct((NC, K), jnp.float32),
        ],
        scratch_types=[
            pltpu.VMEM((NCH, CH), jnp.int32),
            pltpu.VMEM((BPW, D), jnp.float32),
            pltpu.VMEM((CH,), jnp.float32),
            pltpu.VMEM((K // NS,), jnp.float32),
            pltpu.VMEM_SHARED((K,), jnp.float32),
            pltpu.SemaphoreType.DMA,
        ],
        compiler_params=pltpu.CompilerParams(use_tc_tiling_on_sc=False),
    )


def _fin_body(zn_ref, zq_ref, cnt_ref, zste_ref, com_ref, cod_ref, per_ref,
              util_ref):
    zn = zn_ref[...]
    zq = zq_ref[...]
    zste_ref[...] = zn + (zq - zn)
    dlt = zn - zq
    mse = jnp.sum(dlt * dlt) * (1.0 / (B * D))
    com_ref[0, 0] = 0.25 * mse
    cod_ref[0, 0] = mse
    c = cnt_ref[...]
    cc = c[0:1, :] + c[1:2, :]
    p = cc * (1.0 / B)
    ent = -jnp.sum(p * jnp.log(p + 1e-10))
    per_ref[0, 0] = jnp.exp(ent)
    util_ref[0, 0] = jnp.sum((cc > 0.0).astype(jnp.float32)) * (1.0 / K)


def _make_fin(interpret=False):
    scalar = jax.ShapeDtypeStruct((1, 1), jnp.float32)
    return pl.pallas_call(
        _fin_body,
        grid=(1,),
        in_specs=[
            pl.BlockSpec((B, D), lambda i: (0, 0)),
            pl.BlockSpec((B, D), lambda i: (0, 0)),
            pl.BlockSpec((NC, K), lambda i: (0, 0)),
        ],
        out_specs=[
            pl.BlockSpec((B, D), lambda i: (0, 0)),
            pl.BlockSpec((1, 1), lambda i: (0, 0), memory_space=pltpu.SMEM),
            pl.BlockSpec((1, 1), lambda i: (0, 0), memory_space=pltpu.SMEM),
            pl.BlockSpec((1, 1), lambda i: (0, 0), memory_space=pltpu.SMEM),
            pl.BlockSpec((1, 1), lambda i: (0, 0), memory_space=pltpu.SMEM),
        ],
        out_shape=[
            jax.ShapeDtypeStruct((B, D), jnp.float32),
            scalar, scalar, scalar, scalar,
        ],
        interpret=interpret,
    )


_prep = _make_prep()
_main = _make_main()
_fin = _make_fin()
_get_sc = functools.cache(_make_sc)


def kernel(z_e, embeddings):
    en, etn = _prep(embeddings, embeddings.T)
    zn, idx3 = _main(z_e, etn)
    idx2 = idx3.reshape(B // CH, CH)
    zq, cnt2 = _get_sc()(en, idx2)
    zste, com, cod, per, util = _fin(zn, zq, cnt2)
    return (zste, idx3.reshape(B), com.reshape(()), cod.reshape(()),
            per.reshape(()), util.reshape(()))
